# Initial kernel scaffold; baseline (speedup 1.0000x reference)
import jax
import jax.numpy as jnp
from jax.experimental import pallas as pl


def kernel(x, edge_index, W1l, b1l, W1r, W2l, b2l, W2r):
    raise NotImplementedError("write your pallas kernel here")


# speedup vs baseline: 4.0008x; 4.0008x over previous
import jax
import jax.numpy as jnp
from jax import lax
from jax.experimental import pallas as pl
from jax.experimental.pallas import tpu as pltpu
from jax.experimental.pallas import tpu_sc as plsc

N_NODES = 10000
N_EDGES = 320000
D = 128

NC = 2
NS = 16
NW = NC * NS

CHUNK = 64
CHUNKS_PER_TILE = 160
EDGES_PER_TILE = CHUNK * CHUNKS_PER_TILE
E_PAD = EDGES_PER_TILE * NW
N_CHUNK_ROWS = E_PAD // CHUNK
N_PAD = 10240
NPR = N_PAD // 128
ROWS_PER_TILE = N_PAD // NS


def _agg_body(x_hbm, src_hbm, dst_hbm, out_hbm, cntout_hbm,
              src_v, dst_v, rows0, rows1, z16, cnt_v,
              acc, sem0, sem1):
    c = lax.axis_index("c")
    s = lax.axis_index("s")
    wid = s * NC + c

    base = wid * CHUNKS_PER_TILE
    pltpu.sync_copy(src_hbm.at[pl.ds(base, CHUNKS_PER_TILE)], src_v)
    pltpu.sync_copy(dst_hbm.at[pl.ds(base, CHUNKS_PER_TILE)], dst_v)

    def fill_z16(i, carry):
        for cc in range(D // 16):
            z16[i, pl.ds(cc * 16, 16)] = jnp.zeros((16,), jnp.float32)
        return carry
    lax.fori_loop(0, 16, fill_z16, 0)

    def zero_cnt(i, carry):
        for cc in range(D // 16):
            cnt_v[i, pl.ds(cc * 16, 16)] = jnp.zeros((16,), jnp.float32)
        return carry
    lax.fori_loop(0, NPR, zero_cnt, 0)

    row0 = s * ROWS_PER_TILE

    def zero_acc(i, carry):
        pltpu.sync_copy(z16, acc.at[pl.ds(row0 + i * 16, 16)])
        return carry
    lax.fori_loop(0, ROWS_PER_TILE // 16, zero_acc, 0)

    plsc.subcore_barrier()

    def issue(j, buf, sem):
        pltpu.async_copy(x_hbm.at[src_v.at[j]], buf, sem)

    def drain(buf, sem):
        pltpu.make_async_copy(x_hbm.at[pl.ds(0, CHUNK)], buf, sem).wait()

    def scatter(j, buf):
        pltpu.sync_copy(buf, acc.at[dst_v.at[j]], add=True)

    ones16 = jnp.ones((16,), jnp.float32)

    def count(j):
        for cc in range(CHUNK // 16):
            idx = dst_v[j, pl.ds(cc * 16, 16)]
            plsc.addupdate_scatter(
                cnt_v, [lax.shift_right_logical(idx, 7),
                        lax.bitwise_and(idx, 127)], ones16)

    issue(0, rows0, sem0)

    def edge_loop(jj, carry):
        j = jj * 2
        issue(j + 1, rows1, sem1)
        drain(rows0, sem0)
        scatter(j, rows0)
        count(j)

        @pl.when(jj < CHUNKS_PER_TILE // 2 - 1)
        def _():
            issue(j + 2, rows0, sem0)

        drain(rows1, sem1)
        scatter(j + 1, rows1)
        count(j + 1)
        return carry
    lax.fori_loop(0, CHUNKS_PER_TILE // 2, edge_loop, 0)

    plsc.subcore_barrier()

    pltpu.sync_copy(acc.at[pl.ds(row0, ROWS_PER_TILE)],
                    out_hbm.at[c, pl.ds(row0, ROWS_PER_TILE)])
    pltpu.sync_copy(cnt_v, cntout_hbm.at[wid])


_agg = pl.kernel(
    _agg_body,
    out_type=[
        jax.ShapeDtypeStruct((NC, N_PAD, D), jnp.float32),
        jax.ShapeDtypeStruct((NW, NPR, 128), jnp.float32),
    ],
    compiler_params=pltpu.CompilerParams(
        use_tc_tiling_on_sc=False, needs_layout_passes=False),
    mesh=plsc.VectorSubcoreMesh(core_axis_name="c", subcore_axis_name="s"),
    scratch_types=[
        pltpu.VMEM((CHUNKS_PER_TILE, CHUNK), jnp.int32),
        pltpu.VMEM((CHUNKS_PER_TILE, CHUNK), jnp.int32),
        pltpu.VMEM((CHUNK, D), jnp.float32),
        pltpu.VMEM((CHUNK, D), jnp.float32),
        pltpu.VMEM((16, D), jnp.float32),
        pltpu.VMEM((NPR, 128), jnp.float32),
        pltpu.VMEM_SHARED((N_PAD, D), jnp.float32),
        pltpu.SemaphoreType.DMA,
        pltpu.SemaphoreType.DMA,
    ],
)


_BLK = 2048
_CB = _BLK // 128


def _dense_body(p0, p1, cn, xr, wl, bl, wr, o):
    rcp = 1.0 / jnp.maximum(cn[...], 1.0)
    mean = (p0[...] + p1[...]) * rcp
    acc = jnp.dot(mean, wl[...], preferred_element_type=jnp.float32)
    acc = acc + jnp.dot(xr[...], wr[...], preferred_element_type=jnp.float32)
    o[...] = jnp.maximum(acc + bl[...], 0.0)


def _dense(sums, cnts, x, Wl, bl, Wr):
    return pl.pallas_call(
        _dense_body,
        grid=(N_PAD // _BLK,),
        in_specs=[
            pl.BlockSpec((None, _BLK, D), lambda i: (0, i, 0)),
            pl.BlockSpec((None, _BLK, D), lambda i: (1, i, 0)),
            pl.BlockSpec((_BLK, 1), lambda i: (i, 0)),
            pl.BlockSpec((_BLK, D), lambda i: (i, 0)),
            pl.BlockSpec((D, D), lambda i: (0, 0)),
            pl.BlockSpec((1, D), lambda i: (0, 0)),
            pl.BlockSpec((D, D), lambda i: (0, 0)),
        ],
        out_specs=pl.BlockSpec((_BLK, D), lambda i: (i, 0)),
        out_shape=jax.ShapeDtypeStruct((N_PAD, D), jnp.float32),
    )(sums, sums, cnts, x, Wl.T, bl.reshape(1, D), Wr.T)


def kernel(x, edge_index, W1l, b1l, W1r, W2l, b2l, W2r):
    src = edge_index[0].astype(jnp.int32)
    dst = edge_index[1].astype(jnp.int32)
    pad = E_PAD - N_EDGES
    src2d = jnp.concatenate(
        [src, jnp.zeros((pad,), jnp.int32)]).reshape(N_CHUNK_ROWS, CHUNK)
    dst2d = jnp.concatenate(
        [dst, jnp.full((pad,), N_NODES, jnp.int32)]).reshape(N_CHUNK_ROWS, CHUNK)
    x_pad = jnp.concatenate(
        [x, jnp.zeros((N_PAD - N_NODES, D), jnp.float32)])

    sums1, cnts1 = _agg(x_pad, src2d, dst2d)
    cnt_col1 = jnp.sum(cnts1, axis=0).reshape(N_PAD, 1)
    hid = _dense(sums1, cnt_col1, x_pad, W1l, b1l, W1r)
    sums2, cnts2 = _agg(hid, src2d, dst2d)
    cnt_col2 = jnp.sum(cnts2, axis=0).reshape(N_PAD, 1)
    out = _dense(sums2, cnt_col2, hid, W2l, b2l, W2r)
    return out[:N_NODES]


# speedup vs baseline: 11.2652x; 2.8158x over previous
import jax
import jax.numpy as jnp
from jax import lax
from jax.experimental import pallas as pl
from jax.experimental.pallas import tpu as pltpu
from jax.experimental.pallas import tpu_sc as plsc

N_NODES = 10000
N_EDGES = 320000
D = 128

NC = 2
NS = 16
NW = NC * NS

CHUNK = 64
CPT0 = 160
CPT1 = 160
CPT_MAX = max(CPT0, CPT1)
E_PAD = CHUNK * NS * (CPT0 + CPT1)
N_CHUNK_ROWS = E_PAD // CHUNK
N_ROWS_ALLOC = N_CHUNK_ROWS + CPT_MAX
E_ALLOC = N_ROWS_ALLOC * CHUNK
N_PAD = 10240
NPR = N_PAD // 128
ROWS_PER_TILE = N_PAD // NS


def _agg_body(x_hbm, src_hbm, dst_hbm, out_hbm, cntout_hbm,
              src_v, dst_v, rows0, rows1, z16, cnt_v,
              acc, sem0, sem1):
    c = lax.axis_index("c")
    s = lax.axis_index("s")
    wid = s * NC + c

    my_cpt = jnp.where(c == 0, CPT0, CPT1)
    base = jnp.where(c == 0, s * CPT0, NS * CPT0 + s * CPT1)
    def load_idx(i, carry):
        pltpu.sync_copy(src_hbm.at[pl.ds(base + i * 32, 32)],
                        src_v.at[pl.ds(i * 32, 32)])
        pltpu.sync_copy(dst_hbm.at[pl.ds(base + i * 32, 32)],
                        dst_v.at[pl.ds(i * 32, 32)])
        return carry
    lax.fori_loop(0, CPT_MAX // 32, load_idx, 0)

    def fill_z16(i, carry):
        for cc in range(D // 16):
            z16[i, pl.ds(cc * 16, 16)] = jnp.zeros((16,), jnp.float32)
        return carry
    lax.fori_loop(0, 16, fill_z16, 0)

    def zero_cnt(i, carry):
        for cc in range(D // 16):
            cnt_v[i, pl.ds(cc * 16, 16)] = jnp.zeros((16,), jnp.float32)
        return carry
    lax.fori_loop(0, NPR, zero_cnt, 0)

    row0 = s * ROWS_PER_TILE

    def zero_acc(i, carry):
        pltpu.sync_copy(z16, acc.at[pl.ds(row0 + i * 16, 16)])
        return carry
    lax.fori_loop(0, ROWS_PER_TILE // 16, zero_acc, 0)

    plsc.subcore_barrier()

    def issue(j, buf, sem):
        pltpu.async_copy(x_hbm.at[src_v.at[j]], buf, sem)

    def drain(buf, sem):
        pltpu.make_async_copy(x_hbm.at[pl.ds(0, CHUNK)], buf, sem).wait()

    def scatter(j, buf):
        pltpu.sync_copy(buf, acc.at[dst_v.at[j]], add=True)

    ones16 = jnp.ones((16,), jnp.float32)

    def count(j):
        for cc in range(CHUNK // 16):
            idx = dst_v[j, pl.ds(cc * 16, 16)]
            plsc.addupdate_scatter(
                cnt_v, [lax.shift_right_logical(idx, 7),
                        lax.bitwise_and(idx, 127)], ones16)

    half = my_cpt // 2

    @pl.when(half > 0)
    def _():
        issue(0, rows0, sem0)

    def edge_loop(jj, carry):
        j = jj * 2
        issue(j + 1, rows1, sem1)
        drain(rows0, sem0)
        scatter(j, rows0)
        count(j)

        @pl.when(jj < half - 1)
        def _():
            issue(j + 2, rows0, sem0)

        drain(rows1, sem1)
        scatter(j + 1, rows1)
        count(j + 1)
        return carry
    lax.fori_loop(0, half, edge_loop, 0)

    plsc.subcore_barrier()

    pltpu.sync_copy(acc.at[pl.ds(row0, ROWS_PER_TILE)],
                    out_hbm.at[c, pl.ds(row0, ROWS_PER_TILE)])
    pltpu.sync_copy(cnt_v, cntout_hbm.at[wid])


_agg = pl.kernel(
    _agg_body,
    out_type=[
        jax.ShapeDtypeStruct((NC, N_PAD, D), jnp.float32),
        jax.ShapeDtypeStruct((NW, NPR, 128), jnp.float32),
    ],
    compiler_params=pltpu.CompilerParams(
        use_tc_tiling_on_sc=False, needs_layout_passes=False),
    mesh=plsc.VectorSubcoreMesh(core_axis_name="c", subcore_axis_name="s"),
    scratch_types=[
        pltpu.VMEM((CPT_MAX, CHUNK), jnp.int32),
        pltpu.VMEM((CPT_MAX, CHUNK), jnp.int32),
        pltpu.VMEM((CHUNK, D), jnp.float32),
        pltpu.VMEM((CHUNK, D), jnp.float32),
        pltpu.VMEM((16, D), jnp.float32),
        pltpu.VMEM((NPR, 128), jnp.float32),
        pltpu.VMEM_SHARED((N_PAD, D), jnp.float32),
        pltpu.SemaphoreType.DMA,
        pltpu.SemaphoreType.DMA,
    ],
)


_BLK = 2048
_CB = _BLK // 128


def _dense_body(p0, p1, cn, xr, wl, bl, wr, o):
    rcp = 1.0 / jnp.maximum(cn[...], 1.0)
    mean = (p0[...] + p1[...]) * rcp
    acc = jnp.dot(mean, wl[...], preferred_element_type=jnp.float32)
    acc = acc + jnp.dot(xr[...], wr[...], preferred_element_type=jnp.float32)
    o[...] = jnp.maximum(acc + bl[...], 0.0)


def _dense(sums, cnts, x, Wl, bl, Wr):
    return pl.pallas_call(
        _dense_body,
        grid=(N_PAD // _BLK,),
        in_specs=[
            pl.BlockSpec((None, _BLK, D), lambda i: (0, i, 0)),
            pl.BlockSpec((None, _BLK, D), lambda i: (1, i, 0)),
            pl.BlockSpec((_BLK, 1), lambda i: (i, 0)),
            pl.BlockSpec((_BLK, D), lambda i: (i, 0)),
            pl.BlockSpec((D, D), lambda i: (0, 0)),
            pl.BlockSpec((1, D), lambda i: (0, 0)),
            pl.BlockSpec((D, D), lambda i: (0, 0)),
        ],
        out_specs=pl.BlockSpec((_BLK, D), lambda i: (i, 0)),
        out_shape=jax.ShapeDtypeStruct((N_PAD, D), jnp.float32),
    )(sums, sums, cnts, x, Wl.T, bl.reshape(1, D), Wr.T)


def kernel(x, edge_index, W1l, b1l, W1r, W2l, b2l, W2r):
    src = edge_index[0].astype(jnp.int32)
    dst = edge_index[1].astype(jnp.int32)
    pad = E_ALLOC - N_EDGES
    pad_src = (jnp.arange(pad, dtype=jnp.int32) * 61) % N_NODES
    pad_dst = N_NODES + (jnp.arange(pad, dtype=jnp.int32) % (N_PAD - N_NODES))
    src2d = jnp.concatenate([src, pad_src]).reshape(N_ROWS_ALLOC, CHUNK)
    dst2d = jnp.concatenate([dst, pad_dst]).reshape(N_ROWS_ALLOC, CHUNK)
    x_pad = jnp.concatenate(
        [x, jnp.zeros((N_PAD - N_NODES, D), jnp.float32)])

    sums1, cnts1 = _agg(x_pad, src2d, dst2d)
    cnt_col1 = jnp.sum(cnts1, axis=0).reshape(N_PAD, 1)
    hid = _dense(sums1, cnt_col1, x_pad, W1l, b1l, W1r)
    sums2, cnts2 = _agg(hid, src2d, dst2d)
    cnt_col2 = jnp.sum(cnts2, axis=0).reshape(N_PAD, 1)
    out = _dense(sums2, cnt_col2, hid, W2l, b2l, W2r)
    return out[:N_NODES]


# speedup vs baseline: 13.4694x; 1.1957x over previous
import jax
import jax.numpy as jnp
from jax import lax
from jax.experimental import pallas as pl
from jax.experimental.pallas import tpu as pltpu
from jax.experimental.pallas import tpu_sc as plsc

N_NODES = 10000
N_EDGES = 320000
D = 128

NC = 2
NS = 16
NW = NC * NS

CHUNK = 64
CPT = 160
SEG = 32
NSEG = CPT // SEG
E_PAD = CHUNK * CPT * NW
N_CHUNK_ROWS = E_PAD // CHUNK
N_PAD = 10240
NPR = N_PAD // 128
ROWS_PER_TILE = N_PAD // NS


def _agg_body(x_hbm, ed_hbm, out_hbm, cntout_hbm,
              seg, rows0, rows1, rows2, cnt_v, acc,
              gsem0, gsem1, gsem2, ssem0, ssem1, ssem2, isem):
    c = lax.axis_index("c")
    s = lax.axis_index("s")
    wid = s * NC + c
    base = wid * CPT

    rows = (rows0, rows1, rows2)
    gsem = (gsem0, gsem1, gsem2)
    ssem = (ssem0, ssem1, ssem2)

    def zero_rows0(i, carry):
        for cc in range(D // 16):
            rows0[i, pl.ds(cc * 16, 16)] = jnp.zeros((16,), jnp.float32)
        return carry
    lax.fori_loop(0, CHUNK, zero_rows0, 0)

    def zero_cnt(i, carry):
        for cc in range(D // 16):
            cnt_v[i, pl.ds(cc * 16, 16)] = jnp.zeros((16,), jnp.float32)
        return carry
    lax.fori_loop(0, NPR, zero_cnt, 0)

    row0 = s * ROWS_PER_TILE

    def zero_acc(i, carry):
        pltpu.sync_copy(rows0, acc.at[pl.ds(row0 + i * CHUNK, CHUNK)])
        return carry
    lax.fori_loop(0, ROWS_PER_TILE // CHUNK, zero_acc, 0)

    plsc.subcore_barrier()

    pltpu.sync_copy(ed_hbm.at[pl.ds(base, SEG)], seg.at[0])

    def src_idx(q):
        return seg.at[(q // SEG) % 2, lax.rem(q, SEG), 0]

    def dst_idx(q):
        return seg.at[(q // SEG) % 2, lax.rem(q, SEG), 1]

    def issue(q, b):
        pltpu.async_copy(x_hbm.at[src_idx(q)], rows[b], gsem[b])

    def drain_g(b):
        pltpu.make_async_copy(x_hbm.at[pl.ds(0, CHUNK)], rows[b], gsem[b]).wait()

    def scatter(q, b):
        pltpu.async_copy(rows[b], acc.at[dst_idx(q)], ssem[b], add=True)

    def drain_s(b):
        pltpu.make_async_copy(rows[b], acc.at[pl.ds(0, CHUNK)], ssem[b]).wait()

    ones16 = jnp.ones((16,), jnp.float32)

    def count(q):
        a = (q // SEG) % 2
        r = lax.rem(q, SEG)
        for cc in range(CHUNK // 16):
            idx = seg[a, r, 1, pl.ds(cc * 16, 16)]
            plsc.addupdate_scatter(
                cnt_v, [lax.shift_right_logical(idx, 7),
                        lax.bitwise_and(idx, 127)], ones16)

    def step(q, b, first):
        nb = (b + 2) % 3
        k1 = q // SEG + 1

        if first:
            @pl.when(q >= 1)
            def _():
                drain_s(nb)
        else:
            drain_s(nb)

        @pl.when(jnp.logical_and(lax.rem(q, SEG) == 0, k1 < NSEG))
        def _():
            pltpu.async_copy(ed_hbm.at[pl.ds(base + k1 * SEG, SEG)],
                             seg.at[lax.rem(k1, 2)], isem)

        @pl.when(jnp.logical_and(lax.rem(q, SEG) == SEG - 2, k1 < NSEG))
        def _():
            pltpu.make_async_copy(ed_hbm.at[pl.ds(0, SEG)], seg.at[0],
                                  isem).wait()

        @pl.when(q + 2 < CPT)
        def _():
            issue(q + 2, nb)

        drain_g(b)
        scatter(q, b)
        count(q)

    issue(0, 0)
    issue(1, 1)

    def group(g, carry):
        step(g * 3, 0, True)
        step(g * 3 + 1, 1, False)
        step(g * 3 + 2, 2, False)
        return carry
    lax.fori_loop(0, (CPT - 1) // 3, group, 0)

    qt = CPT - 1
    drain_g(0)
    scatter(qt, 0)
    count(qt)
    drain_s(2)
    drain_s(0)

    plsc.subcore_barrier()

    pltpu.sync_copy(acc.at[pl.ds(row0, ROWS_PER_TILE)],
                    out_hbm.at[c, pl.ds(row0, ROWS_PER_TILE)])
    pltpu.sync_copy(cnt_v, cntout_hbm.at[wid])


_agg = pl.kernel(
    _agg_body,
    out_type=[
        jax.ShapeDtypeStruct((NC, N_PAD, D), jnp.float32),
        jax.ShapeDtypeStruct((NW, NPR, 128), jnp.float32),
    ],
    compiler_params=pltpu.CompilerParams(
        use_tc_tiling_on_sc=False, needs_layout_passes=False),
    mesh=plsc.VectorSubcoreMesh(core_axis_name="c", subcore_axis_name="s"),
    scratch_types=[
        pltpu.VMEM((2, SEG, 2, CHUNK), jnp.int32),
        pltpu.VMEM((CHUNK, D), jnp.float32),
        pltpu.VMEM((CHUNK, D), jnp.float32),
        pltpu.VMEM((CHUNK, D), jnp.float32),
        pltpu.VMEM((NPR, 128), jnp.float32),
        pltpu.VMEM_SHARED((N_PAD, D), jnp.float32),
        pltpu.SemaphoreType.DMA,
        pltpu.SemaphoreType.DMA,
        pltpu.SemaphoreType.DMA,
        pltpu.SemaphoreType.DMA,
        pltpu.SemaphoreType.DMA,
        pltpu.SemaphoreType.DMA,
        pltpu.SemaphoreType.DMA,
    ],
)


_BLK = 2048


def _dense_body(p0, p1, cn, xr, wl, bl, wr, o):
    rcp = 1.0 / jnp.maximum(cn[...], 1.0)
    mean = (p0[...] + p1[...]) * rcp
    acc = jnp.dot(mean, wl[...], preferred_element_type=jnp.float32)
    acc = acc + jnp.dot(xr[...], wr[...], preferred_element_type=jnp.float32)
    o[...] = jnp.maximum(acc + bl[...], 0.0)


def _dense(sums, cnts, x, Wl, bl, Wr):
    return pl.pallas_call(
        _dense_body,
        grid=(N_PAD // _BLK,),
        in_specs=[
            pl.BlockSpec((None, _BLK, D), lambda i: (0, i, 0)),
            pl.BlockSpec((None, _BLK, D), lambda i: (1, i, 0)),
            pl.BlockSpec((_BLK, 1), lambda i: (i, 0)),
            pl.BlockSpec((_BLK, D), lambda i: (i, 0)),
            pl.BlockSpec((D, D), lambda i: (0, 0)),
            pl.BlockSpec((1, D), lambda i: (0, 0)),
            pl.BlockSpec((D, D), lambda i: (0, 0)),
        ],
        out_specs=pl.BlockSpec((_BLK, D), lambda i: (i, 0)),
        out_shape=jax.ShapeDtypeStruct((N_PAD, D), jnp.float32),
    )(sums, sums, cnts, x, Wl.T, bl.reshape(1, D), Wr.T)


def kernel(x, edge_index, W1l, b1l, W1r, W2l, b2l, W2r):
    src = edge_index[0].astype(jnp.int32)
    dst = edge_index[1].astype(jnp.int32)
    pad = E_PAD - N_EDGES
    pad_src = (jnp.arange(pad, dtype=jnp.int32) * 61) % N_NODES
    pad_dst = N_NODES + (jnp.arange(pad, dtype=jnp.int32) % (N_PAD - N_NODES))
    src2d = jnp.concatenate([src, pad_src]).reshape(N_CHUNK_ROWS, CHUNK)
    dst2d = jnp.concatenate([dst, pad_dst]).reshape(N_CHUNK_ROWS, CHUNK)
    ed2d = jnp.stack([src2d, dst2d], axis=1)
    x_pad = jnp.concatenate(
        [x, jnp.zeros((N_PAD - N_NODES, D), jnp.float32)])

    sums1, cnts1 = _agg(x_pad, ed2d)
    cnt_col1 = jnp.sum(cnts1, axis=0).reshape(N_PAD, 1)
    hid = _dense(sums1, cnt_col1, x_pad, W1l, b1l, W1r)
    sums2, cnts2 = _agg(hid, ed2d)
    cnt_col2 = jnp.sum(cnts2, axis=0).reshape(N_PAD, 1)
    out = _dense(sums2, cnt_col2, hid, W2l, b2l, W2r)
    return out[:N_NODES]


# speedup vs baseline: 13.9392x; 1.0349x over previous
import jax
import jax.numpy as jnp
from jax import lax
from jax.experimental import pallas as pl
from jax.experimental.pallas import tpu as pltpu
from jax.experimental.pallas import tpu_sc as plsc

N_NODES = 10000
N_EDGES = 320000
D = 128

NC = 2
NS = 16
NW = NC * NS

CHUNK = 64
CPT = 160
SEG = 32
NSEG = CPT // SEG
E_PAD = CHUNK * CPT * NW
N_CHUNK_ROWS = E_PAD // CHUNK
N_PAD = 10240
NPR = N_PAD // 128
ROWS_PER_TILE = N_PAD // NS


def _agg_body(x_hbm, ed_hbm, out_hbm, cntout_hbm,
              seg, rows0, rows1, rows2, cnt_v, acc,
              gsem0, gsem1, gsem2, ssem0, ssem1, ssem2, isem):
    c = lax.axis_index("c")
    s = lax.axis_index("s")
    wid = s * NC + c
    base = wid * CPT

    rows = (rows0, rows1, rows2)
    gsem = (gsem0, gsem1, gsem2)
    ssem = (ssem0, ssem1, ssem2)

    def zero_rows0(i, carry):
        for cc in range(D // 16):
            rows0[i, pl.ds(cc * 16, 16)] = jnp.zeros((16,), jnp.float32)
        return carry
    lax.fori_loop(0, CHUNK, zero_rows0, 0)

    def zero_cnt(i, carry):
        for cc in range(D // 16):
            cnt_v[i, pl.ds(cc * 16, 16)] = jnp.zeros((16,), jnp.float32)
        return carry
    lax.fori_loop(0, NPR, zero_cnt, 0)

    row0 = s * ROWS_PER_TILE

    def zero_acc(i, carry):
        pltpu.sync_copy(rows0, acc.at[pl.ds(row0 + i * CHUNK, CHUNK)])
        return carry
    lax.fori_loop(0, ROWS_PER_TILE // CHUNK, zero_acc, 0)

    plsc.subcore_barrier()

    pltpu.sync_copy(ed_hbm.at[pl.ds(base, SEG)], seg.at[0])

    def src_idx(q):
        return seg.at[(q // SEG) % 2, lax.rem(q, SEG), 0]

    def dst_idx(q):
        return seg.at[(q // SEG) % 2, lax.rem(q, SEG), 1]

    def issue(q, b):
        pltpu.async_copy(x_hbm.at[src_idx(q)], rows[b], gsem[b])

    def drain_g(b):
        pltpu.make_async_copy(x_hbm.at[pl.ds(0, CHUNK)], rows[b], gsem[b]).wait()

    def scatter(q, b):
        pltpu.async_copy(rows[b], acc.at[dst_idx(q)], ssem[b], add=True)

    def drain_s(b):
        pltpu.make_async_copy(rows[b], acc.at[pl.ds(0, CHUNK)], ssem[b]).wait()

    ones16 = jnp.ones((16,), jnp.float32)

    def count(q):
        a = (q // SEG) % 2
        r = lax.rem(q, SEG)
        for cc in range(CHUNK // 16):
            idx = seg[a, r, 1, pl.ds(cc * 16, 16)]
            plsc.addupdate_scatter(
                cnt_v, [lax.shift_right_logical(idx, 7),
                        lax.bitwise_and(idx, 127)], ones16)

    def step(q, b, first):
        nb = (b + 2) % 3
        k1 = q // SEG + 1

        if first:
            @pl.when(q >= 1)
            def _():
                drain_s(nb)
        else:
            drain_s(nb)

        @pl.when(jnp.logical_and(lax.rem(q, SEG) == 0, k1 < NSEG))
        def _():
            pltpu.async_copy(ed_hbm.at[pl.ds(base + k1 * SEG, SEG)],
                             seg.at[lax.rem(k1, 2)], isem)

        @pl.when(jnp.logical_and(lax.rem(q, SEG) == SEG - 2, k1 < NSEG))
        def _():
            pltpu.make_async_copy(ed_hbm.at[pl.ds(0, SEG)], seg.at[0],
                                  isem).wait()

        @pl.when(q + 2 < CPT)
        def _():
            issue(q + 2, nb)

        drain_g(b)
        scatter(q, b)
        count(q)

    issue(0, 0)
    issue(1, 1)

    def group(g, carry):
        step(g * 3, 0, True)
        step(g * 3 + 1, 1, False)
        step(g * 3 + 2, 2, False)
        return carry
    lax.fori_loop(0, (CPT - 1) // 3, group, 0)

    qt = CPT - 1
    drain_g(0)
    scatter(qt, 0)
    count(qt)
    drain_s(2)
    drain_s(0)

    plsc.subcore_barrier()

    pltpu.sync_copy(acc.at[pl.ds(row0, ROWS_PER_TILE)],
                    out_hbm.at[c, pl.ds(row0, ROWS_PER_TILE)])
    pltpu.sync_copy(cnt_v, cntout_hbm.at[wid])


_agg = pl.kernel(
    _agg_body,
    out_type=[
        jax.ShapeDtypeStruct((NC, N_PAD, D), jnp.float32),
        jax.ShapeDtypeStruct((NW, NPR, 128), jnp.float32),
    ],
    compiler_params=pltpu.CompilerParams(
        use_tc_tiling_on_sc=False, needs_layout_passes=False),
    mesh=plsc.VectorSubcoreMesh(core_axis_name="c", subcore_axis_name="s"),
    scratch_types=[
        pltpu.VMEM((2, SEG, 2, CHUNK), jnp.int32),
        pltpu.VMEM((CHUNK, D), jnp.float32),
        pltpu.VMEM((CHUNK, D), jnp.float32),
        pltpu.VMEM((CHUNK, D), jnp.float32),
        pltpu.VMEM((NPR, 128), jnp.float32),
        pltpu.VMEM_SHARED((N_PAD, D), jnp.float32),
        pltpu.SemaphoreType.DMA,
        pltpu.SemaphoreType.DMA,
        pltpu.SemaphoreType.DMA,
        pltpu.SemaphoreType.DMA,
        pltpu.SemaphoreType.DMA,
        pltpu.SemaphoreType.DMA,
        pltpu.SemaphoreType.DMA,
    ],
)


_BLK = 2048


def _dense_body(p0, p1, cn, xr, wl, bl, wr, o):
    rcp = 1.0 / jnp.maximum(cn[...], 1.0)
    mean = (p0[...] + p1[...]) * rcp
    acc = jnp.dot(mean, wl[...], preferred_element_type=jnp.float32)
    acc = acc + jnp.dot(xr[...], wr[...], preferred_element_type=jnp.float32)
    o[...] = jnp.maximum(acc + bl[...], 0.0)


def _dense(sums, cnts, x, Wl, bl, Wr):
    return pl.pallas_call(
        _dense_body,
        grid=(N_PAD // _BLK,),
        in_specs=[
            pl.BlockSpec((None, _BLK, D), lambda i: (0, i, 0)),
            pl.BlockSpec((None, _BLK, D), lambda i: (1, i, 0)),
            pl.BlockSpec((_BLK, 1), lambda i: (i, 0)),
            pl.BlockSpec((_BLK, D), lambda i: (i, 0)),
            pl.BlockSpec((D, D), lambda i: (0, 0)),
            pl.BlockSpec((1, D), lambda i: (0, 0)),
            pl.BlockSpec((D, D), lambda i: (0, 0)),
        ],
        out_specs=pl.BlockSpec((_BLK, D), lambda i: (i, 0)),
        out_shape=jax.ShapeDtypeStruct((N_NODES, D), jnp.float32),
    )(sums, sums, cnts, x, Wl.T, bl.reshape(1, D), Wr.T)


def kernel(x, edge_index, W1l, b1l, W1r, W2l, b2l, W2r):
    src = edge_index[0].astype(jnp.int32)
    dst = edge_index[1].astype(jnp.int32)
    pad = E_PAD - N_EDGES
    pad_src = (jnp.arange(pad, dtype=jnp.int32) * 61) % N_NODES
    pad_dst = N_NODES + (jnp.arange(pad, dtype=jnp.int32) % (N_PAD - N_NODES))
    src2d = jnp.concatenate([src, pad_src]).reshape(N_CHUNK_ROWS, CHUNK)
    dst2d = jnp.concatenate([dst, pad_dst]).reshape(N_CHUNK_ROWS, CHUNK)
    ed2d = jnp.stack([src2d, dst2d], axis=1)

    sums1, cnts1 = _agg(x, ed2d)
    cnt_col1 = jnp.sum(cnts1, axis=0).reshape(N_PAD, 1)
    hid = _dense(sums1, cnt_col1, x, W1l, b1l, W1r)
    sums2, cnts2 = _agg(hid, ed2d)
    cnt_col2 = jnp.sum(cnts2, axis=0).reshape(N_PAD, 1)
    out = _dense(sums2, cnt_col2, hid, W2l, b2l, W2r)
    return out
